# Initial kernel scaffold; baseline (speedup 1.0000x reference)
import jax
import jax.numpy as jnp
from jax.experimental import pallas as pl


def kernel(q_table, expected_q_table):
    raise NotImplementedError("write your pallas kernel here")


# speedup vs baseline: 30.9779x; 30.9779x over previous
import functools

import jax
import jax.numpy as jnp
from jax import lax
from jax.experimental import pallas as pl
from jax.experimental.pallas import tpu as pltpu
from jax.experimental.pallas import tpu_sc as plsc

ETA = 0.5
_R, _N = 128, 32768
_NC, _NS = 2, 16
_NW = _NC * _NS
_RPW = _R // _NW
_L = 16
_BIG = 2**30

_mesh = plsc.VectorSubcoreMesh(core_axis_name="c", subcore_axis_name="s")

_GATHER_DNUMS = lax.GatherDimensionNumbers(
    offset_dims=(), collapsed_slice_dims=(0,), start_index_map=(0,))


def _perm(v, idx):
    return lax.gather(v, idx[:, None], _GATHER_DNUMS, (1,),
                      unique_indices=True, indices_are_sorted=False,
                      mode=lax.GatherScatterMode.PROMISE_IN_BOUNDS)


def _all_reduce(v, op, lanes):
    for sh in (8, 4, 2, 1):
        v = op(v, _perm(v, jnp.bitwise_xor(lanes, sh)))
    return v


@functools.partial(
    pl.kernel,
    out_type=jax.ShapeDtypeStruct((_NW, _L), jnp.float32),
    mesh=_mesh,
    scratch_types=[
        pltpu.VMEM((_N,), jnp.float32),
        pltpu.VMEM((_N,), jnp.float32),
        pltpu.VMEM((_L,), jnp.float32),
    ],
)
def _camp_partials(q_hbm, e_hbm, out_hbm, qbuf, ebuf, obuf):
    wid = lax.axis_index("s") * _NC + lax.axis_index("c")
    lane = lax.iota(jnp.int32, _L)
    neg_inf = jnp.full((_L,), -jnp.inf, jnp.float32)
    zeros = jnp.zeros((_L,), jnp.float32)
    ones = jnp.ones((_L,), jnp.float32)

    ssum = zeros
    scnt = zeros
    for r in range(_RPW):
        row = wid * _RPW + r
        pltpu.sync_copy(q_hbm.at[row], qbuf)
        pltpu.sync_copy(e_hbm.at[row], ebuf)

        def step(t, carry):
            m1, m2, i1, em, ei, idx = carry
            base = pl.multiple_of(t * _L, _L)
            x = qbuf[pl.ds(base, _L)]
            e = ebuf[pl.ds(base, _L)]
            gt = x > m1
            m2 = jnp.maximum(m2, jnp.minimum(x, m1))
            m1 = jnp.maximum(m1, x)
            i1 = jnp.where(gt, idx, i1)
            ge = e > em
            em = jnp.maximum(em, e)
            ei = jnp.where(ge, idx, ei)
            return m1, m2, i1, em, ei, idx + _L

        init = (neg_inf, neg_inf, lane, neg_inf, lane, lane)
        m1, m2, i1, em, ei, _ = lax.fori_loop(
            0, _N // _L, step, init, unroll=8)

        mv = _all_reduce(m1, jnp.maximum, lane)
        candi = jnp.where(m1 == mv, i1, _BIG)
        i1v = _all_reduce(candi, jnp.minimum, lane)
        m1ex = jnp.where(candi == i1v, neg_inf, m1)
        secv = _all_reduce(jnp.maximum(m1ex, m2), jnp.maximum, lane)
        emv = _all_reduce(em, jnp.maximum, lane)
        cande = jnp.where(em == emv, ei, _BIG)
        eiv = _all_reduce(cande, jnp.minimum, lane)

        gapv = secv - mv
        selv = (i1v == eiv) & (jnp.abs(gapv) <= ETA)
        ssum = ssum + jnp.where(selv, gapv + ETA, zeros)
        scnt = scnt + jnp.where(selv, ones, zeros)

    obuf[...] = jnp.where(lane == 0, ssum, jnp.where(lane == 1, scnt, zeros))
    pltpu.sync_copy(obuf, out_hbm.at[wid])


def kernel(q_table, expected_q_table):
    partials = _camp_partials(q_table, expected_q_table)
    s = jnp.sum(partials[:, 0])
    c = jnp.sum(partials[:, 1])
    return jnp.where(c > 0, s / jnp.maximum(c, 1.0), 0.0)


# speedup vs baseline: 38.8613x; 1.2545x over previous
import functools

import jax
import jax.numpy as jnp
from jax import lax
from jax.experimental import pallas as pl
from jax.experimental.pallas import tpu as pltpu
from jax.experimental.pallas import tpu_sc as plsc

ETA = 0.5
_R, _N = 128, 32768
_NC, _NS = 2, 16
_NW = _NC * _NS
_RPW = _R // _NW
_L = 16
_BIG = 2**30
_CH = 8192
_CPR = _N // _CH
_NCHUNK = _RPW * _CPR

_mesh = plsc.VectorSubcoreMesh(core_axis_name="c", subcore_axis_name="s")

_GATHER_DNUMS = lax.GatherDimensionNumbers(
    offset_dims=(), collapsed_slice_dims=(0,), start_index_map=(0,))


def _perm(v, idx):
    return lax.gather(v, idx[:, None], _GATHER_DNUMS, (1,),
                      unique_indices=True, indices_are_sorted=False,
                      mode=lax.GatherScatterMode.PROMISE_IN_BOUNDS)


def _all_reduce(v, op, lanes):
    for sh in (8, 4, 2, 1):
        v = op(v, _perm(v, jnp.bitwise_xor(lanes, sh)))
    return v


@functools.partial(
    pl.kernel,
    out_type=jax.ShapeDtypeStruct((_NW, _L), jnp.float32),
    mesh=_mesh,
    scratch_types=[
        pltpu.VMEM((_CH,), jnp.float32),
        pltpu.VMEM((_CH,), jnp.float32),
        pltpu.VMEM((_CH,), jnp.float32),
        pltpu.VMEM((_CH,), jnp.float32),
        pltpu.VMEM((_L,), jnp.float32),
        pltpu.SemaphoreType.DMA,
        pltpu.SemaphoreType.DMA,
        pltpu.SemaphoreType.DMA,
        pltpu.SemaphoreType.DMA,
    ],
)
def _camp_partials(q_hbm, e_hbm, out_hbm, qbuf0, qbuf1, ebuf0, ebuf1, obuf,
                   sq0, sq1, se0, se1):
    wid = lax.axis_index("s") * _NC + lax.axis_index("c")
    lane = lax.iota(jnp.int32, _L)
    neg_inf = jnp.full((_L,), -jnp.inf, jnp.float32)
    zeros = jnp.zeros((_L,), jnp.float32)
    ones = jnp.ones((_L,), jnp.float32)
    qbufs, ebufs = (qbuf0, qbuf1), (ebuf0, ebuf1)
    qsems, esems = (sq0, sq1), (se0, se1)

    def copies(k):
        r, c = divmod(k, _CPR)
        slot = k % 2
        row = wid * _RPW + r
        qc = pltpu.make_async_copy(
            q_hbm.at[row, pl.ds(c * _CH, _CH)], qbufs[slot], qsems[slot])
        ec = pltpu.make_async_copy(
            e_hbm.at[row, pl.ds(c * _CH, _CH)], ebufs[slot], esems[slot])
        return qc, ec

    def start(k):
        qc, ec = copies(k)
        qc.start()
        ec.start()

    ssum = zeros
    scnt = zeros
    st = None
    start(0)
    for k in range(_NCHUNK):
        r, c = divmod(k, _CPR)
        slot = k % 2
        if k + 1 < _NCHUNK:
            start(k + 1)
        qc, ec = copies(k)
        qc.wait()
        ec.wait()
        qb, eb = qbufs[slot], ebufs[slot]

        if c == 0:
            st = (neg_inf, neg_inf, lane, neg_inf, lane,
                  neg_inf, neg_inf, lane + _L, neg_inf, lane + _L,
                  lane)

        def step(t, carry, qb=qb, eb=eb, c=c):
            (m1a, m2a, i1a, ema, eia,
             m1b, m2b, i1b, emb, eib, idxa) = carry
            base = pl.multiple_of(t * (2 * _L), 2 * _L)
            xa = qb[pl.ds(base, _L)]
            xb = qb[pl.ds(base + _L, _L)]
            ya = eb[pl.ds(base, _L)]
            yb = eb[pl.ds(base + _L, _L)]
            idxb = idxa + _L

            gta = xa > m1a
            m2a = jnp.maximum(m2a, jnp.minimum(xa, m1a))
            m1a = jnp.maximum(m1a, xa)
            i1a = jnp.where(gta, idxa, i1a)
            gea = ya > ema
            ema = jnp.maximum(ema, ya)
            eia = jnp.where(gea, idxa, eia)

            gtb = xb > m1b
            m2b = jnp.maximum(m2b, jnp.minimum(xb, m1b))
            m1b = jnp.maximum(m1b, xb)
            i1b = jnp.where(gtb, idxb, i1b)
            geb = yb > emb
            emb = jnp.maximum(emb, yb)
            eib = jnp.where(geb, idxb, eib)

            return (m1a, m2a, i1a, ema, eia,
                    m1b, m2b, i1b, emb, eib, idxa + 2 * _L)

        st = lax.fori_loop(0, _CH // (2 * _L), step, st, unroll=4)

        if c == _CPR - 1:
            (m1a, m2a, i1a, ema, eia,
             m1b, m2b, i1b, emb, eib, _) = st
            gtab = m1a > m1b
            eqab = m1a == m1b
            m1 = jnp.maximum(m1a, m1b)
            i1 = jnp.where(gtab, i1a,
                           jnp.where(eqab, jnp.minimum(i1a, i1b), i1b))
            m2 = jnp.maximum(jnp.maximum(m2a, m2b), jnp.minimum(m1a, m1b))
            geab = ema > emb
            eqe = ema == emb
            em = jnp.maximum(ema, emb)
            ei = jnp.where(geab, eia,
                           jnp.where(eqe, jnp.minimum(eia, eib), eib))

            mv = _all_reduce(m1, jnp.maximum, lane)
            candi = jnp.where(m1 == mv, i1, _BIG)
            i1v = _all_reduce(candi, jnp.minimum, lane)
            m1ex = jnp.where(candi == i1v, neg_inf, m1)
            secv = _all_reduce(jnp.maximum(m1ex, m2), jnp.maximum, lane)
            emv = _all_reduce(em, jnp.maximum, lane)
            cande = jnp.where(em == emv, ei, _BIG)
            eiv = _all_reduce(cande, jnp.minimum, lane)

            gapv = secv - mv
            selv = (i1v == eiv) & (jnp.abs(gapv) <= ETA)
            ssum = ssum + jnp.where(selv, gapv + ETA, zeros)
            scnt = scnt + jnp.where(selv, ones, zeros)

    obuf[...] = jnp.where(lane == 0, ssum, jnp.where(lane == 1, scnt, zeros))
    pltpu.sync_copy(obuf, out_hbm.at[wid])


def kernel(q_table, expected_q_table):
    partials = _camp_partials(q_table, expected_q_table)
    s = jnp.sum(partials[:, 0])
    c = jnp.sum(partials[:, 1])
    return jnp.where(c > 0, s / jnp.maximum(c, 1.0), 0.0)
